# Initial kernel scaffold; baseline (speedup 1.0000x reference)
import jax
import jax.numpy as jnp
from jax.experimental import pallas as pl


def kernel(time_indices, table):
    raise NotImplementedError("write your pallas kernel here")


# speedup vs baseline: 6.3392x; 6.3392x over previous
import functools

import jax
import jax.numpy as jnp
from jax import lax
from jax.experimental import pallas as pl
from jax.experimental.pallas import tpu as pltpu
from jax.experimental.pallas import tpu_sc as plsc

EMB = 32
PER_DMA = 128
K = 16
CHUNK = K * PER_DMA
NW = 32


@functools.partial(jax.jit, static_argnums=(2,))
def _lookup(idx2d, table, total):
    per_w = total // NW
    groups = per_w // CHUNK
    mesh = plsc.VectorSubcoreMesh(core_axis_name="c", subcore_axis_name="s")

    @functools.partial(
        pl.kernel,
        out_type=jax.ShapeDtypeStruct((total, EMB), jnp.float32),
        mesh=mesh,
        scratch_types=[
            pltpu.VMEM((K, PER_DMA), jnp.int32),
            pltpu.VMEM((CHUNK, EMB), jnp.float32),
            pltpu.SemaphoreType.DMA,
        ],
        compiler_params=pltpu.CompilerParams(use_tc_tiling_on_sc=False),
    )
    def body(table_hbm, idx_hbm, out_hbm, idx_v, rows_v, sem):
        wid = lax.axis_index("s") * 2 + lax.axis_index("c")
        row_base = wid * (per_w // PER_DMA)
        out_base = wid * per_w

        def step(g, carry):
            pltpu.sync_copy(idx_hbm.at[pl.ds(row_base + g * K, K)], idx_v)
            copies = [
                pltpu.async_copy(
                    table_hbm.at[idx_v.at[j]],
                    rows_v.at[pl.ds(j * PER_DMA, PER_DMA)],
                    sem,
                )
                for j in range(K)
            ]
            for c in copies:
                c.wait()
            pltpu.sync_copy(
                rows_v, out_hbm.at[pl.ds(out_base + g * CHUNK, CHUNK)]
            )
            return carry

        lax.fori_loop(0, groups, step, 0)

    return body(table, idx2d)


def kernel(time_indices, table):
    b, t = time_indices.shape
    total = b * t
    idx2d = time_indices.reshape(total // PER_DMA, PER_DMA)
    out = _lookup(idx2d, table, total)
    return out.reshape(b, t, EMB)
